# baseline (speedup 1.0000x reference)
import functools

import jax
import jax.numpy as jnp
from jax import lax
from jax.experimental import pallas as pl
from jax.experimental.pallas import tpu as pltpu
from jax.experimental.pallas import tpu_sc as plsc

N_NODE = 100000
DIM = 128
LAMBDA_DIS = 1e-05
B = 500000

NC = 2
NS = 16
NW = NC * NS
LANES = 16
CB = 128
NCH0 = 190
NCH1 = 58
EPW0 = CB * NCH0
EPW1 = CB * NCH1
B_PAD = NS * (EPW0 + EPW1)
K = DIM // LANES
R = B_PAD // 128

_GATHER_DNUMS = lax.GatherDimensionNumbers(
    offset_dims=(), collapsed_slice_dims=(0,), start_index_map=(0,))


def _permute(v, idx):
    return lax.gather(v, idx[:, None], _GATHER_DNUMS, slice_sizes=(1,),
                      mode=lax.GatherScatterMode.PROMISE_IN_BOUNDS)


def _sc_scores_body(table, nid_h, nbr_h, bias_h,
                    scores_o, sq_o,
                    idxA, idxB,
                    rows1a, rows2a, biasa, scoresa,
                    rows1b, rows2b, biasb, scoresb,
                    sq_v,
                    g1a, g2a, g3a, g1b, g2b, g3b, osema, osemb):
    c = lax.axis_index("c")
    s = lax.axis_index("s")
    wid = s * NC + c
    on_core0 = c == 0
    nch = jnp.where(on_core0, NCH0, NCH1)
    base = jnp.where(on_core0, s * EPW0, NS * EPW0 + s * EPW1)
    lane = lax.iota(jnp.int32, LANES)

    @pl.when(on_core0)
    def _():
        pltpu.sync_copy(nid_h.at[pl.ds(base, EPW0)], idxA)
        pltpu.sync_copy(nbr_h.at[pl.ds(base, EPW0)], idxB)

    @pl.when(jnp.logical_not(on_core0))
    def _():
        pltpu.sync_copy(nid_h.at[pl.ds(base, EPW1)], idxA.at[pl.ds(0, EPW1)])
        pltpu.sync_copy(nbr_h.at[pl.ds(base, EPW1)], idxB.at[pl.ds(0, EPW1)])

    def start_gathers(c, rows1x, rows2x, biasx, s1, s2, s3):
        ia = idxA.at[pl.ds(c * CB, CB)]
        ib = idxB.at[pl.ds(c * CB, CB)]
        pltpu.make_async_copy(table.at[ia], rows1x, s1).start()
        pltpu.make_async_copy(table.at[ib], rows2x, s2).start()
        pltpu.make_async_copy(bias_h.at[ib], biasx, s3).start()

    def wait_gathers(c, rows1x, rows2x, biasx, s1, s2, s3):
        ia = idxA.at[pl.ds(c * CB, CB)]
        ib = idxB.at[pl.ds(c * CB, CB)]
        pltpu.make_async_copy(table.at[ia], rows1x, s1).wait()
        pltpu.make_async_copy(table.at[ib], rows2x, s2).wait()
        pltpu.make_async_copy(bias_h.at[ib], biasx, s3).wait()

    start_gathers(0, rows1a, rows2a, biasa, g1a, g2a, g3a)
    start_gathers(1, rows1b, rows2b, biasb, g1b, g2b, g3b)

    def compute_chunk(rows1x, rows2x, biasx, scoresx, sq):
        def group(g, sqg):
            sq1 = list(sqg[:K])
            sq2 = list(sqg[K:2 * K])
            bsq = sqg[2 * K]
            bvec = biasx[pl.ds(g * LANES, LANES)]
            bsq = bsq + bvec * bvec
            block = jnp.zeros((LANES,), jnp.float32)
            for p in range(LANES):
                e = g * LANES + p
                acc0 = None
                acc1 = None
                for k in range(K // 2):
                    w1 = rows1x[e, pl.ds(k * LANES, LANES)]
                    w2 = rows2x[e, pl.ds(k * LANES, LANES)]
                    v1a = plsc.bitcast(w1 << 16, jnp.float32)
                    v1b = plsc.bitcast(w1 & jnp.int32(-65536), jnp.float32)
                    v2a = plsc.bitcast(w2 << 16, jnp.float32)
                    v2b = plsc.bitcast(w2 & jnp.int32(-65536), jnp.float32)
                    acc0 = (v1a * v2a if acc0 is None else acc0 + v1a * v2a)
                    acc1 = (v1b * v2b if acc1 is None else acc1 + v1b * v2b)
                    sq1[2 * k] = sq1[2 * k] + v1a * v1a
                    sq1[2 * k + 1] = sq1[2 * k + 1] + v1b * v1b
                    sq2[2 * k] = sq2[2 * k] + v2a * v2a
                    sq2[2 * k + 1] = sq2[2 * k + 1] + v2b * v2b
                v = acc0 + acc1
                for sh in (8, 4, 2, 1):
                    v = v + _permute(v, lane ^ sh)
                block = jnp.where(lane == p, v, block)
            scoresx[pl.ds(g * LANES, LANES)] = block + bvec
            return tuple(sq1) + tuple(sq2) + (bsq,)

        return lax.fori_loop(0, CB // LANES, group, sq)

    def pair(i, sq):
        ca = 2 * i
        cb = 2 * i + 1
        wait_gathers(ca, rows1a, rows2a, biasa, g1a, g2a, g3a)

        @pl.when(i > 0)
        def _():
            pltpu.make_async_copy(
                scoresa, scores_o.at[pl.ds(base, CB)], osema).wait()

        sq = compute_chunk(rows1a, rows2a, biasa, scoresa, sq)

        @pl.when(ca + 2 < nch)
        def _():
            start_gathers(ca + 2, rows1a, rows2a, biasa, g1a, g2a, g3a)

        pltpu.make_async_copy(
            scoresa, scores_o.at[pl.ds(base + ca * CB, CB)], osema).start()

        wait_gathers(cb, rows1b, rows2b, biasb, g1b, g2b, g3b)

        @pl.when(i > 0)
        def _():
            pltpu.make_async_copy(
                scoresb, scores_o.at[pl.ds(base, CB)], osemb).wait()

        sq = compute_chunk(rows1b, rows2b, biasb, scoresb, sq)

        @pl.when(cb + 2 < nch)
        def _():
            start_gathers(cb + 2, rows1b, rows2b, biasb, g1b, g2b, g3b)

        pltpu.make_async_copy(
            scoresb, scores_o.at[pl.ds(base + cb * CB, CB)], osemb).start()

        return sq

    sq0 = tuple(jnp.zeros((LANES,), jnp.float32) for _ in range(2 * K + 1))
    sq = lax.fori_loop(0, nch // 2, pair, sq0)

    pltpu.make_async_copy(scoresa, scores_o.at[pl.ds(base, CB)], osema).wait()
    pltpu.make_async_copy(scoresb, scores_o.at[pl.ds(base, CB)], osemb).wait()

    total = sq[0]
    for t in sq[1:]:
        total = total + t
    sq_v[...] = total
    pltpu.sync_copy(sq_v, sq_o.at[wid])


_sc_scores = functools.partial(
    pl.kernel,
    mesh=plsc.VectorSubcoreMesh(core_axis_name="c", subcore_axis_name="s"),
    compiler_params=pltpu.CompilerParams(
        needs_layout_passes=False, use_tc_tiling_on_sc=False),
    out_type=[
        jax.ShapeDtypeStruct((B_PAD,), jnp.float32),
        jax.ShapeDtypeStruct((NW, LANES), jnp.float32),
    ],
    scratch_types=[
        pltpu.VMEM((EPW0,), jnp.int32),
        pltpu.VMEM((EPW0,), jnp.int32),
        pltpu.VMEM((CB, DIM // 2), jnp.int32),
        pltpu.VMEM((CB, DIM // 2), jnp.int32),
        pltpu.VMEM((CB,), jnp.float32),
        pltpu.VMEM((CB,), jnp.float32),
        pltpu.VMEM((CB, DIM // 2), jnp.int32),
        pltpu.VMEM((CB, DIM // 2), jnp.int32),
        pltpu.VMEM((CB,), jnp.float32),
        pltpu.VMEM((CB,), jnp.float32),
        pltpu.VMEM((LANES,), jnp.float32),
        pltpu.SemaphoreType.DMA,
        pltpu.SemaphoreType.DMA,
        pltpu.SemaphoreType.DMA,
        pltpu.SemaphoreType.DMA,
        pltpu.SemaphoreType.DMA,
        pltpu.SemaphoreType.DMA,
        pltpu.SemaphoreType.DMA,
        pltpu.SemaphoreType.DMA,
    ],
)(_sc_scores_body)


def _tc_combine_body(scores, label, sq, emb0, bias0, out):
    s = scores[...]
    y = label[...].astype(jnp.float32)
    pos = (lax.broadcasted_iota(jnp.int32, (R, 128), 0) * 128
           + lax.broadcasted_iota(jnp.int32, (R, 128), 1))
    validf = (pos < B).astype(jnp.float32)
    prob = jax.nn.sigmoid(s)
    eps = 1e-12
    ll = (y * jnp.log(jnp.clip(prob, eps, 1.0))
          + (1.0 - y) * jnp.log(jnp.clip(1.0 - prob, eps, 1.0)))
    bce = -jnp.sum(ll * validf) / B
    e0 = emb0[...]
    col0 = (lax.broadcasted_iota(jnp.int32, (1, 128), 1) == 0)
    b0 = bias0[...] * col0.astype(jnp.float32)
    corr = float(B_PAD - B) * (2.0 * jnp.sum(e0 * e0) + jnp.sum(b0 * b0))
    l2 = jnp.sum(sq[...]) - corr
    total = bce + l2 * (0.5 * LAMBDA_DIS)
    out[...] = jnp.reshape(total, (1, 1))


def kernel(node_id, node_neighbor_id, label, embedding_matrix, bias):
    pad = B_PAD - B
    zi = jnp.zeros((pad,), jnp.int32)
    nid = jnp.concatenate([node_id, zi])
    nbr = jnp.concatenate([node_neighbor_id, zi])
    lab = jnp.concatenate([label, zi])
    table_b = embedding_matrix.astype(jnp.bfloat16)
    table_i = lax.bitcast_convert_type(
        table_b.reshape(N_NODE, DIM // 2, 2), jnp.int32)
    scores, sq = _sc_scores(table_i, nid, nbr, bias)
    emb0 = table_b[0:1, :].astype(jnp.float32)
    bias0 = bias[0:128].reshape(1, 128)
    out = pl.pallas_call(
        _tc_combine_body,
        out_shape=jax.ShapeDtypeStruct((1, 1), jnp.float32),
    )(scores.reshape(R, 128), lab.reshape(R, 128), sq, emb0, bias0)
    return out[0, 0]
